# baseline (speedup 1.0000x reference)
import functools

import numpy as np
import jax
import jax.numpy as jnp
from jax import lax
from jax.experimental import pallas as pl
from jax.experimental.pallas import tpu as pltpu
from jax.experimental.pallas import tpu_sc as plsc

LEVELS = 16
CHANNELS = 2
TABLE_SIZE = 1 << 15
BASE_RES = 16
GROWTH = 1.5
IN_COORDS = 3
HIDDEN = 64
OUT_DIM = 5
BATCH = 131072
ENC_DIM = IN_COORDS * LEVELS * CHANNELS
NPAIR = IN_COORDS * LEVELS
NWORKERS = 32
UNITS_PER_W = 3
HALF = BATCH // 2
CHUNK = 4096
NCHUNK = HALF // CHUNK
NVEC = CHUNK // 16
HASH_K = -1640531535


def _level_consts():
    scales, res = [], []
    for l in range(LEVELS):
        s = float(2.0 ** (l * np.log2(GROWTH)) * BASE_RES - 1.0)
        r = int(np.ceil(s)) + 1
        scales.append(s)
        res.append(r)
    return np.array(scales, np.float32), np.array(res, np.int32)


_SCALES, _RES = _level_consts()


def _make_encoder():
    mesh = plsc.VectorSubcoreMesh(core_axis_name="c", subcore_axis_name="s")

    @functools.partial(
        pl.kernel,
        mesh=mesh,
        out_type=jax.ShapeDtypeStruct((ENC_DIM * BATCH,), jnp.float32),
        compiler_params=pltpu.CompilerParams(needs_layout_passes=False),
        scratch_types=[
            pltpu.VMEM((TABLE_SIZE * CHANNELS,), jnp.float32),
            pltpu.VMEM((CHUNK * 6,), jnp.float32),
            pltpu.VMEM((CHUNK,), jnp.float32),
            pltpu.VMEM((CHUNK,), jnp.float32),
            pltpu.VMEM((LEVELS,), jnp.float32),
            pltpu.VMEM((LEVELS,), jnp.int32),
        ],
    )
    def encode(xf, tab, scales, resa, out, table_v, xb, o0, o1, sc_v, rs_v):
        pltpu.sync_copy(scales, sc_v)
        pltpu.sync_copy(resa, rs_v)
        wid = lax.axis_index("s") * 2 + lax.axis_index("c")
        iota6 = lax.iota(jnp.int32, 16) * 6
        for u in range(UNITS_PER_W):
            unit = wid * UNITS_PER_W + u
            pair = unit >> 1
            halfsel = unit & 1
            coord = pair >> 4
            level = pair & 15
            base = halfsel * HALF
            pltpu.sync_copy(tab.at[pl.ds(pair * (TABLE_SIZE * CHANNELS),
                                         TABLE_SIZE * CHANNELS)], table_v)
            lvl_v = jnp.full((16,), level, jnp.int32)
            scale_v = plsc.load_gather(sc_v, [lvl_v])
            res_v = plsc.load_gather(rs_v, [lvl_v])
            resm1 = res_v - 1
            is_hash = (res_v * res_v) > TABLE_SIZE
            xcol = coord * 2
            orow_off = pair * 2 * BATCH + base

            def chunk_body(ci, carry):
                off = ci * CHUNK
                pltpu.sync_copy(xf.at[pl.ds((base + off) * 6, CHUNK * 6)], xb)

                def vec_body(i, carry2):
                    s0 = pl.multiple_of(i * 16, 16)
                    ix = iota6 + (i * 96 + xcol)
                    xv = plsc.load_gather(xb, [ix])
                    yv = plsc.load_gather(xb, [ix + 1])
                    px = xv * scale_v + 0.5
                    py = yv * scale_v + 0.5
                    p0x = px.astype(jnp.int32)
                    p0y = py.astype(jnp.int32)
                    wx = px - p0x.astype(jnp.float32)
                    wy = py - p0y.astype(jnp.float32)
                    cx1 = jnp.minimum(p0x + 1, resm1)
                    cy1 = jnp.minimum(p0y + 1, resm1)
                    wx0 = 1.0 - wx
                    wy0 = 1.0 - wy
                    acc0 = jnp.zeros((16,), jnp.float32)
                    acc1 = jnp.zeros((16,), jnp.float32)
                    for cx, cy, w in (
                        (p0x, p0y, wx0 * wy0),
                        (p0x, cy1, wx0 * wy),
                        (cx1, p0y, wx * wy0),
                        (cx1, cy1, wx * wy),
                    ):
                        direct = cx * res_v + cy
                        hashed = (cx ^ (cy * HASH_K)) & (TABLE_SIZE - 1)
                        fi = jnp.where(is_hash, hashed, direct) * 2
                        acc0 = acc0 + w * plsc.load_gather(table_v, [fi])
                        acc1 = acc1 + w * plsc.load_gather(table_v, [fi + 1])
                    o0[pl.ds(s0, 16)] = acc0
                    o1[pl.ds(s0, 16)] = acc1
                    return carry2

                lax.fori_loop(0, NVEC, vec_body, 0)
                pltpu.sync_copy(o0, out.at[pl.ds(orow_off + off, CHUNK)])
                pltpu.sync_copy(o1, out.at[pl.ds(orow_off + BATCH + off, CHUNK)])
                return carry

            lax.fori_loop(0, NCHUNK, chunk_body, 0)

    return encode


_encode = _make_encoder()


def _mlp(enc_t, w1t, w2t, w3p):
    cb = 1024

    def body(e_ref, w1_ref, w2_ref, w3_ref, o4_ref, o1_ref):
        h = jnp.maximum(
            lax.dot(w1_ref[...], e_ref[...], preferred_element_type=jnp.float32), 0.0)
        h = jnp.maximum(
            lax.dot(w2_ref[...], h, preferred_element_type=jnp.float32), 0.0)
        o = lax.dot_general(h, w3_ref[...], (((0,), (0,)), ((), ())),
                            preferred_element_type=jnp.float32)
        o4_ref[...] = o[:, :4]
        o1_ref[...] = o[:, 4:5]

    return pl.pallas_call(
        body,
        grid=(BATCH // cb,),
        in_specs=[
            pl.BlockSpec((ENC_DIM, cb), lambda i: (0, i)),
            pl.BlockSpec((HIDDEN, ENC_DIM), lambda i: (0, 0)),
            pl.BlockSpec((HIDDEN, HIDDEN), lambda i: (0, 0)),
            pl.BlockSpec((HIDDEN, 8), lambda i: (0, 0)),
        ],
        out_specs=[
            pl.BlockSpec((cb, 4), lambda i: (i, 0)),
            pl.BlockSpec((cb, 1), lambda i: (i, 0)),
        ],
        out_shape=[
            jax.ShapeDtypeStruct((BATCH, 4), jnp.float32),
            jax.ShapeDtypeStruct((BATCH, 1), jnp.float32),
        ],
    )(enc_t, w1t, w2t, w3p)


def kernel(x, tables, W1, W2, W3):
    xf = x.reshape(-1)
    tab = tables.reshape(-1)
    enc_flat = _encode(xf, tab, jnp.asarray(_SCALES), jnp.asarray(_RES))
    enc_t = enc_flat.reshape(ENC_DIM, BATCH)
    w1t = W1.T
    w2t = W2.T
    w3p = jnp.pad(W3, ((0, 0), (0, 3)))
    out4, out1 = _mlp(enc_t, w1t, w2t, w3p)
    return (out4, out1)


# speedup vs baseline: 3.3028x; 3.3028x over previous
import functools

import numpy as np
import jax
import jax.numpy as jnp
from jax import lax
from jax.experimental import pallas as pl
from jax.experimental.pallas import tpu as pltpu
from jax.experimental.pallas import tpu_sc as plsc

LEVELS = 16
CHANNELS = 2
TABLE_SIZE = 1 << 15
BASE_RES = 16
GROWTH = 1.5
IN_COORDS = 3
HIDDEN = 64
OUT_DIM = 5
BATCH = 131072
ENC_DIM = IN_COORDS * LEVELS * CHANNELS
NPAIR = IN_COORDS * LEVELS
NWORKERS = 32
UNITS_PER_W = 3
HALF = BATCH // 2
CHUNK = 4096
NCHUNK = HALF // CHUNK
NVEC = CHUNK // 16
HASH_K = -1640531535


def _level_consts():
    scales, res = [], []
    for l in range(LEVELS):
        s = float(2.0 ** (l * np.log2(GROWTH)) * BASE_RES - 1.0)
        r = int(np.ceil(s)) + 1
        scales.append(s)
        res.append(r)
    return np.array(scales, np.float32), np.array(res, np.int32)


_SCALES, _RES = _level_consts()


def _make_encoder():
    mesh = plsc.VectorSubcoreMesh(core_axis_name="c", subcore_axis_name="s")

    @functools.partial(
        pl.kernel,
        mesh=mesh,
        out_type=jax.ShapeDtypeStruct((ENC_DIM * BATCH,), jnp.float32),
        compiler_params=pltpu.CompilerParams(needs_layout_passes=False),
        scratch_types=[
            pltpu.VMEM((TABLE_SIZE * CHANNELS,), jnp.float32),
            pltpu.VMEM((CHUNK * 6,), jnp.float32),
            pltpu.VMEM((CHUNK,), jnp.float32),
            pltpu.VMEM((CHUNK,), jnp.float32),
            pltpu.VMEM((LEVELS,), jnp.float32),
            pltpu.VMEM((LEVELS,), jnp.int32),
        ],
    )
    def encode(xf, tab, scales, resa, out, table_v, xb, o0, o1, sc_v, rs_v):
        pltpu.sync_copy(scales, sc_v)
        pltpu.sync_copy(resa, rs_v)
        wid = lax.axis_index("s") * 2 + lax.axis_index("c")
        iota6 = lax.iota(jnp.int32, 16) * 6
        for u in range(UNITS_PER_W):
            unit = wid * UNITS_PER_W + u
            pair = unit >> 1
            halfsel = unit & 1
            coord = pair >> 4
            level = pair & 15
            base = halfsel * HALF
            pltpu.sync_copy(tab.at[pl.ds(pair * (TABLE_SIZE * CHANNELS),
                                         TABLE_SIZE * CHANNELS)], table_v)
            lvl_v = jnp.full((16,), level, jnp.int32)
            scale_v = plsc.load_gather(sc_v, [lvl_v])
            res_v = plsc.load_gather(rs_v, [lvl_v])
            resm1 = res_v - 1
            is_hash = (res_v * res_v) > TABLE_SIZE
            xcol = coord * 2
            orow_off = pair * 2 * BATCH + base

            def chunk_body(ci, carry):
                off = ci * CHUNK
                pltpu.sync_copy(xf.at[pl.ds((base + off) * 6, CHUNK * 6)], xb)

                def vec_body(i, carry2):
                    s0 = pl.multiple_of(i * 16, 16)
                    ix = iota6 + (i * 96 + xcol)
                    xv = plsc.load_gather(xb, [ix])
                    yv = plsc.load_gather(xb, [ix + 1])
                    px = xv * scale_v + 0.5
                    py = yv * scale_v + 0.5
                    p0x = px.astype(jnp.int32)
                    p0y = py.astype(jnp.int32)
                    wx = px - p0x.astype(jnp.float32)
                    wy = py - p0y.astype(jnp.float32)
                    cx1 = jnp.minimum(p0x + 1, resm1)
                    cy1 = jnp.minimum(p0y + 1, resm1)
                    wx0 = 1.0 - wx
                    wy0 = 1.0 - wy
                    acc0 = jnp.zeros((16,), jnp.float32)
                    acc1 = jnp.zeros((16,), jnp.float32)
                    for cx, cy, w in (
                        (p0x, p0y, wx0 * wy0),
                        (p0x, cy1, wx0 * wy),
                        (cx1, p0y, wx * wy0),
                        (cx1, cy1, wx * wy),
                    ):
                        direct = cx * res_v + cy
                        hashed = (cx ^ (cy * HASH_K)) & (TABLE_SIZE - 1)
                        idx = jnp.where(is_hash, hashed, direct)
                        fi = idx + (idx & -128)
                        acc0 = acc0 + w * plsc.load_gather(table_v, [fi])
                        acc1 = acc1 + w * plsc.load_gather(table_v, [fi + 128])
                    o0[pl.ds(s0, 16)] = acc0
                    o1[pl.ds(s0, 16)] = acc1
                    return carry2

                lax.fori_loop(0, NVEC, vec_body, 0)
                pltpu.sync_copy(o0, out.at[pl.ds(orow_off + off, CHUNK)])
                pltpu.sync_copy(o1, out.at[pl.ds(orow_off + BATCH + off, CHUNK)])
                return carry

            lax.fori_loop(0, NCHUNK, chunk_body, 0)

    return encode


_encode = _make_encoder()


def _mlp(enc_t, w1t, w2t, w3p):
    cb = 1024

    def body(e_ref, w1_ref, w2_ref, w3_ref, o4_ref, o1_ref):
        h = jnp.maximum(
            lax.dot(w1_ref[...], e_ref[...], preferred_element_type=jnp.float32), 0.0)
        h = jnp.maximum(
            lax.dot(w2_ref[...], h, preferred_element_type=jnp.float32), 0.0)
        o = lax.dot_general(h, w3_ref[...], (((0,), (0,)), ((), ())),
                            preferred_element_type=jnp.float32)
        o4_ref[...] = o[:, :4]
        o1_ref[...] = o[:, 4:5]

    return pl.pallas_call(
        body,
        grid=(BATCH // cb,),
        in_specs=[
            pl.BlockSpec((ENC_DIM, cb), lambda i: (0, i)),
            pl.BlockSpec((HIDDEN, ENC_DIM), lambda i: (0, 0)),
            pl.BlockSpec((HIDDEN, HIDDEN), lambda i: (0, 0)),
            pl.BlockSpec((HIDDEN, 8), lambda i: (0, 0)),
        ],
        out_specs=[
            pl.BlockSpec((cb, 4), lambda i: (i, 0)),
            pl.BlockSpec((cb, 1), lambda i: (i, 0)),
        ],
        out_shape=[
            jax.ShapeDtypeStruct((BATCH, 4), jnp.float32),
            jax.ShapeDtypeStruct((BATCH, 1), jnp.float32),
        ],
    )(enc_t, w1t, w2t, w3p)


def kernel(x, tables, W1, W2, W3):
    xf = x.reshape(-1)
    tab = (tables.transpose(0, 1, 3, 2)
           .reshape(IN_COORDS, LEVELS, CHANNELS, TABLE_SIZE // 128, 128)
           .transpose(0, 1, 3, 2, 4)
           .reshape(-1))
    enc_flat = _encode(xf, tab, jnp.asarray(_SCALES), jnp.asarray(_RES))
    enc_t = enc_flat.reshape(ENC_DIM, BATCH)
    w1t = W1.T
    w2t = W2.T
    w3p = jnp.pad(W3, ((0, 0), (0, 3)))
    out4, out1 = _mlp(enc_t, w1t, w2t, w3p)
    return (out4, out1)
